# baseline (speedup 1.0000x reference)
import jax
import jax.numpy as jnp
from jax.experimental import pallas as pl


def _add_body(x_ref, pe_ref, o_ref):
    o_ref[...] = x_ref[...] + pe_ref[None]


def kernel(x, pos_embedding):
    B, S, D = x.shape
    BS = 512
    return pl.pallas_call(
        _add_body,
        grid=(S // BS, B // 2),
        in_specs=[
            pl.BlockSpec((2, BS, D), lambda s, b: (b, s, 0)),
            pl.BlockSpec((BS, D), lambda s, b: (s, 0)),
        ],
        out_specs=pl.BlockSpec((2, BS, D), lambda s, b: (b, s, 0)),
        out_shape=jax.ShapeDtypeStruct((B, S, D), x.dtype),
    )(x, pos_embedding)


# speedup vs baseline: 1.0076x; 1.0076x over previous
import jax
import jax.numpy as jnp
from jax.experimental import pallas as pl


def _add_body(x_ref, pe_ref, o_ref):
    o_ref[...] = x_ref[...] + pe_ref[None]


def kernel(x, pos_embedding):
    B, S, D = x.shape
    BS = 512
    return pl.pallas_call(
        _add_body,
        grid=(S // BS,),
        in_specs=[
            pl.BlockSpec((B, BS, D), lambda s: (0, s, 0)),
            pl.BlockSpec((BS, D), lambda s: (s, 0)),
        ],
        out_specs=pl.BlockSpec((B, BS, D), lambda s: (0, s, 0)),
        out_shape=jax.ShapeDtypeStruct((B, S, D), x.dtype),
    )(x, pos_embedding)
